# baseline (speedup 1.0000x reference)
import functools

import jax
import jax.numpy as jnp
from jax.experimental import pallas as pl
from jax.experimental.pallas import tpu as pltpu


def _chain_kernel(x_ref, wcat_ref, w1_ref, w2_ref, wfc_ref,
                  bt_ref, b1_ref, bf_ref, out_ref, *, n):
    cur = x_ref[...]
    rowid = jax.lax.broadcasted_iota(jnp.int32, cur.shape, 0)
    xm1 = pltpu.roll(cur, shift=1, axis=0)
    xm1 = jnp.where(rowid == 0, 0.0, xm1)
    xp1 = pltpu.roll(cur, shift=n - 1, axis=0)
    xp1 = jnp.where(rowid == n - 1, 0.0, xp1)
    xin = jnp.concatenate([xm1, cur, xp1], axis=1)
    y = jnp.dot(xin, wcat_ref[...], preferred_element_type=jnp.float32)
    y = jax.nn.relu(y + bt_ref[...])
    z = jnp.dot(y, w1_ref[...], preferred_element_type=jnp.float32)
    z = jax.nn.relu(z + b1_ref[...])
    wf = jnp.dot(w2_ref[...], wfc_ref[...], preferred_element_type=jnp.float32)
    z = jnp.dot(z, wf, preferred_element_type=jnp.float32) + bf_ref[...]
    out_ref[...] = z


def kernel(x, edge_index, edge_weights, W_t, b_t, W1, b1, W2, b2, W_fc, b_fc):
    B, T, N, C = x.shape
    H = W1.shape[0]
    C_OUT = W_fc.shape[1]

    Wcat = jnp.concatenate(
        [W_t[:, :, 0, 0].T, W_t[:, :, 1, 0].T, W_t[:, :, 2, 0].T], axis=0)
    bf = (b2 @ W_fc + b_fc).reshape(1, C_OUT)

    xf = x.reshape(B * T * N, C)

    out = pl.pallas_call(
        functools.partial(_chain_kernel, n=N),
        grid=(B,),
        in_specs=[
            pl.BlockSpec((N, C), lambda k: (k * T + T - 1, 0)),
            pl.BlockSpec((3 * C, H), lambda k: (0, 0)),
            pl.BlockSpec((H, H), lambda k: (0, 0)),
            pl.BlockSpec((H, H), lambda k: (0, 0)),
            pl.BlockSpec((H, C_OUT), lambda k: (0, 0)),
            pl.BlockSpec((1, H), lambda k: (0, 0)),
            pl.BlockSpec((1, H), lambda k: (0, 0)),
            pl.BlockSpec((1, C_OUT), lambda k: (0, 0)),
        ],
        out_specs=pl.BlockSpec((N, C_OUT), lambda k: (k, 0)),
        out_shape=jax.ShapeDtypeStruct((B * N, C_OUT), jnp.float32),
    )(xf, Wcat, W1, W2, W_fc,
      b_t.reshape(1, H), b1.reshape(1, H), bf)
    return out.reshape(B, N, C_OUT)


# speedup vs baseline: 1.8774x; 1.8774x over previous
import functools

import jax
import jax.numpy as jnp
from jax.experimental import pallas as pl
from jax.experimental.pallas import tpu as pltpu

_BATCHES_PER_BLOCK = 1


def _chain_kernel(x_ref, wcat_ref, w1_ref, w2_ref, wfc_ref,
                  bt_ref, b1_ref, bf_ref, out_ref, *, n):
    cur = x_ref[...]
    bm = cur.shape[0]
    rowid = jax.lax.broadcasted_iota(jnp.int32, cur.shape, 0)
    xm1 = pltpu.roll(cur, shift=1, axis=0)
    xm1 = jnp.where(rowid % n == 0, 0.0, xm1)
    xp1 = pltpu.roll(cur, shift=bm - 1, axis=0)
    xp1 = jnp.where(rowid % n == n - 1, 0.0, xp1)
    xin = jnp.concatenate([xm1, cur, xp1], axis=1)
    y = jnp.dot(xin, wcat_ref[...], preferred_element_type=jnp.float32)
    y = jax.nn.relu(y + bt_ref[...])
    z = jnp.dot(y, w1_ref[...], preferred_element_type=jnp.float32)
    z = jax.nn.relu(z + b1_ref[...])
    wf = jnp.dot(w2_ref[...], wfc_ref[...], preferred_element_type=jnp.float32)
    z = jnp.dot(z, wf, preferred_element_type=jnp.float32) + bf_ref[...]
    out_ref[...] = z


def kernel(x, edge_index, edge_weights, W_t, b_t, W1, b1, W2, b2, W_fc, b_fc):
    B, T, N, C = x.shape
    H = W1.shape[0]
    C_OUT = W_fc.shape[1]
    rows = B * N
    bm = _BATCHES_PER_BLOCK * N

    Wcat = jnp.concatenate(
        [W_t[:, :, 0, 0].T, W_t[:, :, 1, 0].T, W_t[:, :, 2, 0].T], axis=0)
    bf = (b2 @ W_fc + b_fc).reshape(1, C_OUT)

    xl = x[:, T - 1].reshape(rows, C)

    out = pl.pallas_call(
        functools.partial(_chain_kernel, n=N),
        grid=(rows // bm,),
        in_specs=[
            pl.BlockSpec((bm, C), lambda k: (k, 0)),
            pl.BlockSpec((3 * C, H), lambda k: (0, 0)),
            pl.BlockSpec((H, H), lambda k: (0, 0)),
            pl.BlockSpec((H, H), lambda k: (0, 0)),
            pl.BlockSpec((H, C_OUT), lambda k: (0, 0)),
            pl.BlockSpec((1, H), lambda k: (0, 0)),
            pl.BlockSpec((1, H), lambda k: (0, 0)),
            pl.BlockSpec((1, C_OUT), lambda k: (0, 0)),
        ],
        out_specs=pl.BlockSpec((bm, C_OUT), lambda k: (k, 0)),
        out_shape=jax.ShapeDtypeStruct((rows, C_OUT), jnp.float32),
    )(xl, Wcat, W1, W2, W_fc,
      b_t.reshape(1, H), b1.reshape(1, H), bf)
    return out.reshape(B, N, C_OUT)
